# baseline (speedup 1.0000x reference)
import functools

import jax
import jax.numpy as jnp
from jax.experimental import pallas as pl
from jax.experimental.pallas import tpu as pltpu


def _vq_tile_kernel(x_ref, e_ref, out_ref, e_norm_ref, e_hi_ref, e_lo_ref,
                    *, num_codes):
    b = pl.program_id(0)
    t = pl.program_id(1)

    @pl.when(jnp.logical_and(b == 0, t == 0))
    def _prologue():
        e = e_ref[...]
        e_norm_ref[...] = jnp.sum(e * e, axis=1, keepdims=True)
        e_hi = e.astype(jnp.bfloat16)
        e_hi_ref[...] = e_hi
        e_lo_ref[...] = (e - e_hi.astype(jnp.float32)).astype(jnp.bfloat16)

    xt = x_ref[0]
    scores = jax.lax.dot_general(
        e_ref[...], xt, (((1,), (0,)), ((), ())),
        preferred_element_type=jnp.float32)
    dist = e_norm_ref[...] - 2.0 * scores
    m = jnp.min(dist, axis=0, keepdims=True)
    k_iota = jax.lax.broadcasted_iota(jnp.int32, dist.shape, 0)
    idx = jnp.min(jnp.where(dist == m, k_iota, num_codes),
                  axis=0, keepdims=True)
    one_hot = (k_iota == idx).astype(jnp.bfloat16)
    q = jax.lax.dot_general(
        e_hi_ref[...], one_hot, (((0,), (0,)), ((), ())),
        preferred_element_type=jnp.float32)
    q += jax.lax.dot_general(
        e_lo_ref[...], one_hot, (((0,), (0,)), ((), ())),
        preferred_element_type=jnp.float32)
    out_ref[0] = q


@functools.partial(jax.jit, static_argnames=("interpret",))
def kernel(x, embeddings, interpret=False):
    B, D, T = x.shape
    K = embeddings.shape[0]
    TT = 256
    quantized = pl.pallas_call(
        functools.partial(_vq_tile_kernel, num_codes=K),
        grid=(B, T // TT),
        in_specs=[
            pl.BlockSpec((1, D, TT), lambda b, t: (b, 0, t)),
            pl.BlockSpec((K, D), lambda b, t: (0, 0)),
        ],
        out_specs=pl.BlockSpec((1, D, TT), lambda b, t: (b, 0, t)),
        out_shape=jax.ShapeDtypeStruct((B, D, T), jnp.float32),
        scratch_shapes=[
            pltpu.VMEM((K, 1), jnp.float32),
            pltpu.VMEM((K, D), jnp.bfloat16),
            pltpu.VMEM((K, D), jnp.bfloat16),
        ],
        interpret=interpret,
    )(x, embeddings)
    return (quantized, x, quantized)


# speedup vs baseline: 1.4651x; 1.4651x over previous
import functools

import jax
import jax.numpy as jnp
from jax.experimental import pallas as pl
from jax.experimental.pallas import tpu as pltpu


def _vq_tile_kernel(x_ref, e_ref, out_ref, h_ref, *, num_codes):
    b = pl.program_id(0)
    t = pl.program_id(1)

    @pl.when(jnp.logical_and(b == 0, t == 0))
    def _prologue():
        e = e_ref[...]
        h_ref[...] = 0.5 * jnp.sum(e * e, axis=1, keepdims=True)

    xt = x_ref[0]
    scores = jax.lax.dot_general(
        e_ref[...], xt, (((1,), (0,)), ((), ())),
        preferred_element_type=jnp.float32)
    score = scores - h_ref[...]
    m = jnp.max(score, axis=0, keepdims=True)
    k_iota = jax.lax.broadcasted_iota(jnp.int32, score.shape, 0)
    idx = jnp.min(jnp.where(score == m, k_iota, num_codes),
                  axis=0, keepdims=True)
    one_hot = (k_iota == idx).astype(jnp.float32)
    q = jax.lax.dot_general(
        e_ref[...], one_hot, (((0,), (0,)), ((), ())),
        preferred_element_type=jnp.float32)
    out_ref[0] = q


@functools.partial(jax.jit, static_argnames=("interpret", "tt"))
def kernel(x, embeddings, interpret=False, tt=512):
    B, D, T = x.shape
    K = embeddings.shape[0]
    quantized = pl.pallas_call(
        functools.partial(_vq_tile_kernel, num_codes=K),
        grid=(B, T // tt),
        in_specs=[
            pl.BlockSpec((1, D, tt), lambda b, t: (b, 0, t)),
            pl.BlockSpec((K, D), lambda b, t: (0, 0)),
        ],
        out_specs=pl.BlockSpec((1, D, tt), lambda b, t: (b, 0, t)),
        out_shape=jax.ShapeDtypeStruct((B, D, T), jnp.float32),
        scratch_shapes=[
            pltpu.VMEM((K, 1), jnp.float32),
        ],
        interpret=interpret,
    )(x, embeddings)
    return (quantized, x, quantized)


# speedup vs baseline: 1.5641x; 1.0676x over previous
import functools

import jax
import jax.numpy as jnp
from jax.experimental import pallas as pl
from jax.experimental.pallas import tpu as pltpu


def _vq_tile_kernel(x_ref, e_ref, out_ref, h_ref, *, num_codes):
    b = pl.program_id(0)
    t = pl.program_id(1)

    @pl.when(jnp.logical_and(b == 0, t == 0))
    def _prologue():
        e = e_ref[...]
        h_ref[...] = jnp.sum(e * e, axis=1, keepdims=True)

    xt = x_ref[0]
    scores = jax.lax.dot_general(
        e_ref[...], xt, (((1,), (0,)), ((), ())),
        preferred_element_type=jnp.float32)
    xnorm = jnp.sum(xt * xt, axis=0, keepdims=True)
    dist = (xnorm + h_ref[...]) - 2.0 * scores
    k_iota = jax.lax.broadcasted_iota(jnp.int32, dist.shape, 0)
    idx = jnp.argmin(dist, axis=0, keepdims=True).astype(jnp.int32)
    one_hot = (k_iota == idx).astype(jnp.float32)
    q = jax.lax.dot_general(
        e_ref[...], one_hot, (((0,), (0,)), ((), ())),
        preferred_element_type=jnp.float32)
    out_ref[0] = q


@functools.partial(jax.jit, static_argnames=("interpret", "tt"))
def kernel(x, embeddings, interpret=False, tt=512):
    B, D, T = x.shape
    K = embeddings.shape[0]
    quantized = pl.pallas_call(
        functools.partial(_vq_tile_kernel, num_codes=K),
        grid=(B, T // tt),
        in_specs=[
            pl.BlockSpec((1, D, tt), lambda b, t: (b, 0, t)),
            pl.BlockSpec((K, D), lambda b, t: (0, 0)),
        ],
        out_specs=pl.BlockSpec((1, D, tt), lambda b, t: (b, 0, t)),
        out_shape=jax.ShapeDtypeStruct((B, D, T), jnp.float32),
        scratch_shapes=[
            pltpu.VMEM((K, 1), jnp.float32),
        ],
        interpret=interpret,
    )(x, embeddings)
    return (quantized, x, quantized)


# speedup vs baseline: 1.8762x; 1.1996x over previous
import functools

import jax
import jax.numpy as jnp
from jax.experimental import pallas as pl
from jax.experimental.pallas import tpu as pltpu


def _vq_tile_kernel(x_ref, e_ref, out_ref, h_ref, *, num_codes):
    b = pl.program_id(0)
    t = pl.program_id(1)

    @pl.when(jnp.logical_and(b == 0, t == 0))
    def _prologue():
        e = e_ref[...]
        h_ref[...] = jnp.sum(e * e, axis=1, keepdims=True)

    xt = x_ref[0]
    scores = jax.lax.dot_general(
        e_ref[...], xt, (((1,), (0,)), ((), ())),
        preferred_element_type=jnp.float32)
    xnorm = jnp.sum(xt * xt, axis=0, keepdims=True)
    dist = (xnorm + h_ref[...]) - 2.0 * scores
    k_iota = jax.lax.broadcasted_iota(jnp.int32, dist.shape, 0)
    idx = jnp.argmin(dist, axis=0, keepdims=True).astype(jnp.int32)
    one_hot = (k_iota == idx).astype(jnp.float32)
    q = jax.lax.dot_general(
        e_ref[...], one_hot, (((0,), (0,)), ((), ())),
        preferred_element_type=jnp.float32)
    out_ref[0] = q


@functools.partial(jax.jit, static_argnames=("interpret", "tt"))
def kernel(x, embeddings, interpret=False, tt=1024):
    B, D, T = x.shape
    K = embeddings.shape[0]
    quantized = pl.pallas_call(
        functools.partial(_vq_tile_kernel, num_codes=K),
        grid=(B, T // tt),
        in_specs=[
            pl.BlockSpec((1, D, tt), lambda b, t: (b, 0, t)),
            pl.BlockSpec((K, D), lambda b, t: (0, 0)),
        ],
        out_specs=pl.BlockSpec((1, D, tt), lambda b, t: (b, 0, t)),
        out_shape=jax.ShapeDtypeStruct((B, D, T), jnp.float32),
        scratch_shapes=[
            pltpu.VMEM((K, 1), jnp.float32),
        ],
        interpret=interpret,
    )(x, embeddings)
    return (quantized, x, quantized)


# speedup vs baseline: 2.1877x; 1.1660x over previous
import functools

import jax
import jax.numpy as jnp
from jax.experimental import pallas as pl
from jax.experimental.pallas import tpu as pltpu


def _vq_tile_kernel(x_ref, e_ref, out_ref, x_out_ref, h_ref, *, num_codes):
    b = pl.program_id(0)
    t = pl.program_id(1)

    @pl.when(jnp.logical_and(b == 0, t == 0))
    def _prologue():
        e = e_ref[...]
        h_ref[...] = jnp.sum(e * e, axis=1, keepdims=True)

    xt = x_ref[0]
    x_out_ref[0] = xt
    scores = jax.lax.dot_general(
        e_ref[...], xt, (((1,), (0,)), ((), ())),
        preferred_element_type=jnp.float32)
    xnorm = jnp.sum(xt * xt, axis=0, keepdims=True)
    dist = (xnorm + h_ref[...]) - 2.0 * scores
    k_iota = jax.lax.broadcasted_iota(jnp.int32, dist.shape, 0)
    idx = jnp.argmin(dist, axis=0, keepdims=True).astype(jnp.int32)
    one_hot = (k_iota == idx).astype(jnp.float32)
    q = jax.lax.dot_general(
        e_ref[...], one_hot, (((0,), (0,)), ((), ())),
        preferred_element_type=jnp.float32)
    out_ref[0] = q


@functools.partial(jax.jit, static_argnames=("interpret", "tt"))
def kernel(x, embeddings, interpret=False, tt=1024):
    B, D, T = x.shape
    K = embeddings.shape[0]
    quantized, x_out = pl.pallas_call(
        functools.partial(_vq_tile_kernel, num_codes=K),
        grid=(B, T // tt),
        in_specs=[
            pl.BlockSpec((1, D, tt), lambda b, t: (b, 0, t)),
            pl.BlockSpec((K, D), lambda b, t: (0, 0)),
        ],
        out_specs=[
            pl.BlockSpec((1, D, tt), lambda b, t: (b, 0, t)),
            pl.BlockSpec((1, D, tt), lambda b, t: (b, 0, t)),
        ],
        out_shape=[
            jax.ShapeDtypeStruct((B, D, T), jnp.float32),
            jax.ShapeDtypeStruct((B, D, T), jnp.float32),
        ],
        scratch_shapes=[
            pltpu.VMEM((K, 1), jnp.float32),
        ],
        interpret=interpret,
    )(x, embeddings)
    return (quantized, x_out, quantized)
